# Initial kernel scaffold; baseline (speedup 1.0000x reference)
import jax
import jax.numpy as jnp
from jax.experimental import pallas as pl


def kernel(x, edge_index, edge_attr, batch, parity_atoms, parity_bond_index, W_edge_init, b_edge_init, conv_W, conv_b, conv_gamma, conv_beta, W_ffn, b_ffn):
    raise NotImplementedError("write your pallas kernel here")


# speedup vs baseline: 2.2965x; 2.2965x over previous
import functools

import jax
import jax.numpy as jnp
from jax import lax
from jax.experimental import pallas as pl
from jax.experimental.pallas import tpu as pltpu
from jax.experimental.pallas import tpu_sc as plsc

N = 10000
E = 160000
H = 64
DN = 48
DE = 13
G = 128
DEPTH = 3

NC = 2
NS = 16
NW = NC * NS
B = 100
NB = (E // NW) // B
NZ = 10
RZ = N // NZ

_mesh = plsc.VectorSubcoreMesh(core_axis_name="c", subcore_axis_name="s",
                               num_cores=NC, num_subcores=NS)
_sc_params = pltpu.CompilerParams(use_tc_tiling_on_sc=False)


@functools.partial(
    pl.kernel,
    out_type=jax.ShapeDtypeStruct((NC, NZ, RZ, H), jnp.float32),
    mesh=_mesh,
    scratch_types=[
        pltpu.VMEM((NB, B), jnp.int32),
        pltpu.VMEM((B, H), jnp.float32),
        pltpu.VMEM_SHARED((N, H), jnp.float32),
    ],
    compiler_params=_sc_params,
)
def _sc_segment_sum(h3, col3, zeros3, out, idx_v, hbuf, acc):
    c = lax.axis_index("c")
    s = lax.axis_index("s")
    wid = c * NS + s

    @pl.when(s < NZ)
    def _zero():
        pltpu.sync_copy(zeros3.at[s], acc.at[pl.ds(s * RZ, RZ)])

    plsc.subcore_barrier()
    pltpu.sync_copy(col3.at[wid], idx_v)

    def body(j, carry):
        pltpu.sync_copy(h3.at[wid * NB + j], hbuf)
        pltpu.sync_copy(hbuf, acc.at[idx_v.at[j]], add=True)
        return carry

    lax.fori_loop(0, NB, body, 0)
    plsc.subcore_barrier()

    @pl.when(s < NZ)
    def _out():
        pltpu.sync_copy(acc.at[pl.ds(s * RZ, RZ)], out.at[c].at[s])


@functools.partial(
    pl.kernel,
    out_type=jax.ShapeDtypeStruct((NW * NB, B, H), jnp.float32),
    mesh=_mesh,
    scratch_types=[
        pltpu.VMEM((NB, B), jnp.int32),
        pltpu.VMEM((B, H), jnp.float32),
        pltpu.SemaphoreType.DMA,
    ],
    compiler_params=_sc_params,
)
def _sc_gather_rows(tab, row3, out, idx_v, buf, sem):
    c = lax.axis_index("c")
    s = lax.axis_index("s")
    wid = c * NS + s
    pltpu.sync_copy(row3.at[wid], idx_v)

    def body(j, carry):
        pltpu.async_copy(tab.at[idx_v.at[j]], buf, sem).wait()
        pltpu.sync_copy(buf, out.at[wid * NB + j])
        return carry

    lax.fori_loop(0, NB, body, 0)


def _mm_body(x_ref, w_ref, o_ref):
    o_ref[...] = jnp.dot(x_ref[...], w_ref[...],
                         preferred_element_type=jnp.float32)


def _node_mm(x, w):
    return pl.pallas_call(
        _mm_body,
        out_shape=jax.ShapeDtypeStruct((N, H), jnp.float32),
    )(x, w)


BE = 8000


def _init_body(xr_ref, ea_ref, we_ref, b_ref, o_ref):
    z = (xr_ref[...]
         + jnp.dot(ea_ref[...], we_ref[...], preferred_element_type=jnp.float32)
         + b_ref[...])
    o_ref[...] = jnp.maximum(z, 0.0)


def _edge_init(xwrow, ea, we, b0):
    return pl.pallas_call(
        _init_body,
        grid=(E // BE,),
        in_specs=[pl.BlockSpec((BE, H), lambda i: (i, 0)),
                  pl.BlockSpec((BE, DE), lambda i: (i, 0)),
                  pl.BlockSpec((DE, H), lambda i: (0, 0)),
                  pl.BlockSpec((1, H), lambda i: (0, 0))],
        out_specs=pl.BlockSpec((BE, H), lambda i: (i, 0)),
        out_shape=jax.ShapeDtypeStruct((E, H), jnp.float32),
    )(xwrow, ea, we, b0)


def _combine_body(p_ref, o_ref):
    o_ref[...] = p_ref[0] + p_ref[1]


def _combine(part):
    return pl.pallas_call(
        _combine_body,
        out_shape=jax.ShapeDtypeStruct((N, H), jnp.float32),
    )(part)


def _layer_body(ar_ref, h_ref, w_ref, b_ref, o_ref):
    h = h_ref[...]
    rows = lax.broadcasted_iota(jnp.int32, (BE, H), 0)
    sw = jnp.where((rows % 2) == 0,
                   jnp.roll(h, -1, axis=0), jnp.roll(h, 1, axis=0))
    m = ar_ref[...] - sw
    z = jnp.dot(m, w_ref[...], preferred_element_type=jnp.float32) + b_ref[...]
    o_ref[...] = 2.0 * jnp.maximum(z, 0.0)


def _layer(arow, h, wp, bp):
    return pl.pallas_call(
        _layer_body,
        grid=(E // BE,),
        in_specs=[pl.BlockSpec((BE, H), lambda i: (i, 0)),
                  pl.BlockSpec((BE, H), lambda i: (i, 0)),
                  pl.BlockSpec((H, H), lambda i: (0, 0)),
                  pl.BlockSpec((1, H), lambda i: (0, 0))],
        out_specs=pl.BlockSpec((BE, H), lambda i: (i, 0)),
        out_shape=jax.ShapeDtypeStruct((E, H), jnp.float32),
    )(arow, h, wp, bp)


BN = 2000


def _pool_body(p_ref, b_ref, wf_ref, bf_ref, o_ref, acc, cnt):
    j = pl.program_id(0)

    @pl.when(j == 0)
    def _():
        acc[...] = jnp.zeros_like(acc)
        cnt[...] = jnp.zeros_like(cnt)

    hn = p_ref[0] + p_ref[1]
    gid = lax.broadcasted_iota(jnp.int32, (BN, G), 1)
    oh = (b_ref[...] == gid).astype(jnp.float32)
    acc[...] += lax.dot_general(oh, hn, (((0,), (0,)), ((), ())),
                                preferred_element_type=jnp.float32)
    cnt[...] += lax.dot_general(oh, jnp.ones((BN, 1), jnp.float32),
                                (((0,), (0,)), ((), ())),
                                preferred_element_type=jnp.float32)

    @pl.when(j == pl.num_programs(0) - 1)
    def _():
        pooled = acc[...] / jnp.maximum(cnt[...], 1.0)
        o_ref[...] = jax.nn.sigmoid(
            jnp.dot(pooled, wf_ref[...], preferred_element_type=jnp.float32)
            + bf_ref[...])


def _pool(part, batch2, wf, bf):
    return pl.pallas_call(
        _pool_body,
        grid=(N // BN,),
        in_specs=[pl.BlockSpec((2, BN, H), lambda i: (0, i, 0)),
                  pl.BlockSpec((BN, 1), lambda i: (i, 0)),
                  pl.BlockSpec((H, 1), lambda i: (0, 0)),
                  pl.BlockSpec((1, 1), lambda i: (0, 0))],
        out_specs=pl.BlockSpec((G, 1), lambda i: (0, 0)),
        out_shape=jax.ShapeDtypeStruct((G, 1), jnp.float32),
        scratch_shapes=[pltpu.VMEM((G, H), jnp.float32),
                        pltpu.VMEM((G, 1), jnp.float32)],
    )(part, batch2, wf, bf)


def kernel(x, edge_index, edge_attr, batch, parity_atoms, parity_bond_index,
           W_edge_init, b_edge_init, conv_W, conv_b, conv_gamma, conv_beta,
           W_ffn, b_ffn):
    row3 = edge_index[0].reshape(NW, NB, B)
    col3 = edge_index[1].reshape(NW, NB, B)
    zeros3 = jnp.zeros((NZ, RZ, H), jnp.float32)

    Wx = W_edge_init[:DN]
    We = W_edge_init[DN:]
    b0 = b_edge_init.reshape(1, H)

    xw = _node_mm(x, Wx)
    xwrow = _sc_gather_rows(xw, row3).reshape(E, H)
    h = _edge_init(xwrow, edge_attr, We, b0)

    for l in range(DEPTH):
        wp = conv_W[l] * conv_gamma[l][None, :]
        bp = (conv_gamma[l] * conv_b[l] + conv_beta[l]).reshape(1, H)
        part = _sc_segment_sum(h.reshape(NW * NB, B, H), col3, zeros3)
        a = _combine(part.reshape(2, N, H))
        arow = _sc_gather_rows(a, row3).reshape(E, H)
        h = _layer(arow, h, wp, bp)

    part = _sc_segment_sum(h.reshape(NW * NB, B, H), col3, zeros3)
    return _pool(part.reshape(2, N, H), batch.reshape(N, 1),
                 W_ffn, b_ffn.reshape(1, 1))


# speedup vs baseline: 2.7264x; 1.1872x over previous
import functools

import jax
import jax.numpy as jnp
from jax import lax
from jax.experimental import pallas as pl
from jax.experimental.pallas import tpu as pltpu
from jax.experimental.pallas import tpu_sc as plsc

N = 10000
E = 160000
H = 64
DN = 48
DE = 13
G = 128
DEPTH = 3

NC = 2
NS = 16
NW = NC * NS
B = 125
NB = (E // NW) // B
LBI = 5
LR = LBI * B
NL = NB // LBI
NZ = 10
RZ = N // NZ

_mesh = plsc.VectorSubcoreMesh(core_axis_name="c", subcore_axis_name="s",
                               num_cores=NC, num_subcores=NS)
_sc_params = pltpu.CompilerParams(use_tc_tiling_on_sc=False)


@functools.partial(
    pl.kernel,
    out_type=jax.ShapeDtypeStruct((NC, NZ, RZ, H), jnp.float32),
    mesh=_mesh,
    scratch_types=[
        pltpu.VMEM((NB, B), jnp.int32),
        pltpu.VMEM((2, LR, H), jnp.float32),
        pltpu.VMEM_SHARED((N, H), jnp.float32),
        pltpu.SemaphoreType.DMA((2,)),
        pltpu.SemaphoreType.DMA((2,)),
    ],
    compiler_params=_sc_params,
)
def _sc_segment_sum(h4, col3, zeros3, out, idx_v, hbuf, acc, lsem, ssem):
    c = lax.axis_index("c")
    s = lax.axis_index("s")
    wid = c * NS + s

    @pl.when(s < NZ)
    def _zero():
        pltpu.sync_copy(zeros3.at[s], acc.at[pl.ds(s * RZ, RZ)])

    pltpu.sync_copy(col3.at[wid], idx_v)
    plsc.subcore_barrier()

    loads = [None, None]
    scats = [[], []]

    def start_load(g):
        b = g % 2
        loads[b] = pltpu.async_copy(h4.at[wid * NL + g], hbuf.at[b],
                                    lsem.at[b])

    start_load(0)
    for g in range(NL):
        b = g % 2
        loads[b].wait()
        for d in scats[1 - b]:
            d.wait()
        scats[1 - b] = []
        if g + 1 < NL:
            start_load(g + 1)
        for j in range(LBI):
            scats[b].append(pltpu.async_copy(
                hbuf.at[b].at[pl.ds(j * B, B)],
                acc.at[idx_v.at[g * LBI + j]], ssem.at[b], add=True))
    for b in range(2):
        for d in scats[b]:
            d.wait()

    plsc.subcore_barrier()

    @pl.when(s < NZ)
    def _out():
        pltpu.sync_copy(acc.at[pl.ds(s * RZ, RZ)], out.at[c].at[s])


@functools.partial(
    pl.kernel,
    out_type=jax.ShapeDtypeStruct((NW * NL, LR, H), jnp.float32),
    mesh=_mesh,
    scratch_types=[
        pltpu.VMEM((NB, B), jnp.int32),
        pltpu.VMEM((2, LR, H), jnp.float32),
        pltpu.SemaphoreType.DMA((2,)),
        pltpu.SemaphoreType.DMA((2,)),
    ],
    compiler_params=_sc_params,
)
def _sc_gather_rows(tab, row3, out, idx_v, buf, gsem, osem):
    c = lax.axis_index("c")
    s = lax.axis_index("s")
    wid = c * NS + s
    pltpu.sync_copy(row3.at[wid], idx_v)

    gaths = [[], []]
    outs = [None, None]

    def fire_gathers(g):
        b = g % 2
        for j in range(LBI):
            gaths[b].append(pltpu.async_copy(
                tab.at[idx_v.at[g * LBI + j]],
                buf.at[b].at[pl.ds(j * B, B)], gsem.at[b]))

    fire_gathers(0)
    for g in range(NL):
        b = g % 2
        for d in gaths[b]:
            d.wait()
        gaths[b] = []
        if g >= 1 and outs[1 - b] is not None:
            outs[1 - b].wait()
        if g + 1 < NL:
            fire_gathers(g + 1)
        outs[b] = pltpu.async_copy(buf.at[b], out.at[wid * NL + g],
                                   osem.at[b])
    outs[(NL - 1) % 2].wait()


def _mm_body(x_ref, w_ref, o_ref):
    o_ref[...] = jnp.dot(x_ref[...], w_ref[...],
                         preferred_element_type=jnp.float32)


def _node_mm(x, w):
    return pl.pallas_call(
        _mm_body,
        out_shape=jax.ShapeDtypeStruct((N, H), jnp.float32),
    )(x, w)


BE = 8000


def _init_body(xr_ref, ea_ref, we_ref, b_ref, o_ref):
    z = (xr_ref[...]
         + jnp.dot(ea_ref[...], we_ref[...], preferred_element_type=jnp.float32)
         + b_ref[...])
    o_ref[...] = jnp.maximum(z, 0.0)


def _edge_init(xwrow, ea, we, b0):
    return pl.pallas_call(
        _init_body,
        grid=(E // BE,),
        in_specs=[pl.BlockSpec((BE, H), lambda i: (i, 0)),
                  pl.BlockSpec((BE, DE), lambda i: (i, 0)),
                  pl.BlockSpec((DE, H), lambda i: (0, 0)),
                  pl.BlockSpec((1, H), lambda i: (0, 0))],
        out_specs=pl.BlockSpec((BE, H), lambda i: (i, 0)),
        out_shape=jax.ShapeDtypeStruct((E, H), jnp.float32),
    )(xwrow, ea, we, b0)


def _combine_body(p_ref, o_ref):
    o_ref[...] = p_ref[0] + p_ref[1]


def _combine(part):
    return pl.pallas_call(
        _combine_body,
        out_shape=jax.ShapeDtypeStruct((N, H), jnp.float32),
    )(part)


def _layer_body(ar_ref, h_ref, w_ref, b_ref, o_ref):
    h = h_ref[...]
    rows = lax.broadcasted_iota(jnp.int32, (BE, H), 0)
    sw = jnp.where((rows % 2) == 0,
                   jnp.roll(h, -1, axis=0), jnp.roll(h, 1, axis=0))
    m = ar_ref[...] - sw
    z = jnp.dot(m, w_ref[...], preferred_element_type=jnp.float32) + b_ref[...]
    o_ref[...] = 2.0 * jnp.maximum(z, 0.0)


def _layer(arow, h, wp, bp):
    return pl.pallas_call(
        _layer_body,
        grid=(E // BE,),
        in_specs=[pl.BlockSpec((BE, H), lambda i: (i, 0)),
                  pl.BlockSpec((BE, H), lambda i: (i, 0)),
                  pl.BlockSpec((H, H), lambda i: (0, 0)),
                  pl.BlockSpec((1, H), lambda i: (0, 0))],
        out_specs=pl.BlockSpec((BE, H), lambda i: (i, 0)),
        out_shape=jax.ShapeDtypeStruct((E, H), jnp.float32),
    )(arow, h, wp, bp)


BN = 2000


def _pool_body(p_ref, b_ref, wf_ref, bf_ref, o_ref, acc, cnt):
    j = pl.program_id(0)

    @pl.when(j == 0)
    def _():
        acc[...] = jnp.zeros_like(acc)
        cnt[...] = jnp.zeros_like(cnt)

    hn = p_ref[0] + p_ref[1]
    gid = lax.broadcasted_iota(jnp.int32, (BN, G), 1)
    oh = (b_ref[...] == gid).astype(jnp.float32)
    acc[...] += lax.dot_general(oh, hn, (((0,), (0,)), ((), ())),
                                preferred_element_type=jnp.float32)
    cnt[...] += lax.dot_general(oh, jnp.ones((BN, 1), jnp.float32),
                                (((0,), (0,)), ((), ())),
                                preferred_element_type=jnp.float32)

    @pl.when(j == pl.num_programs(0) - 1)
    def _():
        pooled = acc[...] / jnp.maximum(cnt[...], 1.0)
        o_ref[...] = jax.nn.sigmoid(
            jnp.dot(pooled, wf_ref[...], preferred_element_type=jnp.float32)
            + bf_ref[...])


def _pool(part, batch2, wf, bf):
    return pl.pallas_call(
        _pool_body,
        grid=(N // BN,),
        in_specs=[pl.BlockSpec((2, BN, H), lambda i: (0, i, 0)),
                  pl.BlockSpec((BN, 1), lambda i: (i, 0)),
                  pl.BlockSpec((H, 1), lambda i: (0, 0)),
                  pl.BlockSpec((1, 1), lambda i: (0, 0))],
        out_specs=pl.BlockSpec((G, 1), lambda i: (0, 0)),
        out_shape=jax.ShapeDtypeStruct((G, 1), jnp.float32),
        scratch_shapes=[pltpu.VMEM((G, H), jnp.float32),
                        pltpu.VMEM((G, 1), jnp.float32)],
    )(part, batch2, wf, bf)


def kernel(x, edge_index, edge_attr, batch, parity_atoms, parity_bond_index,
           W_edge_init, b_edge_init, conv_W, conv_b, conv_gamma, conv_beta,
           W_ffn, b_ffn):
    row3 = edge_index[0].reshape(NW, NB, B)
    col3 = edge_index[1].reshape(NW, NB, B)
    zeros3 = jnp.zeros((NZ, RZ, H), jnp.float32)

    Wx = W_edge_init[:DN]
    We = W_edge_init[DN:]
    b0 = b_edge_init.reshape(1, H)

    xw = _node_mm(x, Wx)
    xwrow = _sc_gather_rows(xw, row3).reshape(E, H)
    h = _edge_init(xwrow, edge_attr, We, b0)

    for l in range(DEPTH):
        wp = conv_W[l] * conv_gamma[l][None, :]
        bp = (conv_gamma[l] * conv_b[l] + conv_beta[l]).reshape(1, H)
        part = _sc_segment_sum(h.reshape(NW * NL, LR, H), col3, zeros3)
        a = _combine(part.reshape(2, N, H))
        arow = _sc_gather_rows(a, row3).reshape(E, H)
        h = _layer(arow, h, wp, bp)

    part = _sc_segment_sum(h.reshape(NW * NL, LR, H), col3, zeros3)
    return _pool(part.reshape(2, N, H), batch.reshape(N, 1),
                 W_ffn, b_ffn.reshape(1, 1))
